# baseline (speedup 1.0000x reference)
import functools

import numpy as np

import jax
import jax.numpy as jnp
from jax.experimental import pallas as pl
from jax.experimental.pallas import tpu as pltpu

NEG = 0.01
EPS = 1e-5
BLK = 8
B = 8


def _band_sel():
    e = np.zeros((3, BLK + 2, BLK), np.float32)
    for kx in range(3):
        for c in range(BLK):
            e[kx, c + kx, c] = 1.0
    return e


def _band_tiles(w_pt):
    cout, cg = w_pt.shape[0], w_pt.shape[1]
    t = jnp.einsum('xrc,oiyx->yrico', _band_sel(), w_pt.astype(jnp.float32))
    return t.reshape(3, (BLK + 2) * cg, BLK * cout).astype(jnp.bfloat16)


def _up_mat(w_pt, w1, cc):
    cm, co = w_pt.shape[0], w_pt.shape[1]
    e2 = np.zeros((2, w1, 2 * w1), np.float32)
    for dj in range(2):
        for j in range(w1):
            e2[dj, j, 2 * j + dj] = 1.0
    wpad = jnp.pad(w_pt.astype(jnp.float32), ((0, 0), (cc - co, 0),
                                              (0, 0), (0, 0)))
    m = jnp.einsum('bjk,ioab->jiako', e2, wpad)
    return m.reshape(w1 * cm, 2 * (2 * w1) * cc).astype(jnp.bfloat16)


def _perm(rows, n_rows, n_src):
    p = np.zeros((n_rows, n_src), np.float32)
    p[np.array(rows), np.arange(n_src)] = 1.0
    return jnp.asarray(p, dtype=jnp.bfloat16)


def _bn_scale_shift(st, w, c, count, gamma, beta):
    s = jnp.sum(st, axis=0).reshape(2, w, c).sum(axis=1)
    mean = s[0] / count
    var = s[1] / count - mean * mean
    scale = gamma * jax.lax.rsqrt(var + EPS)
    shift = beta - mean * scale
    return (jnp.tile(scale, w).reshape(1, w * c),
            jnp.tile(shift, w).reshape(1, w * c))


def _kernel_a(h1, w1, cm, h2, w2, cout, nb,
              x1_ref, x2_ref, um_ref, ub_ref, py_ref, px_ref, w_ref,
              o_ref, st_ref, s_ref):
    f32 = jnp.float32
    bf16 = jnp.bfloat16
    cc = 2 * cm
    m = B * (h2 + 2)
    mp = m + 8
    stride = h2 + 2

    s_ref[0:m, 0:cc] = jnp.zeros((m, cc), bf16)
    s_ref[0:m, cc + w2 * cc:] = jnp.zeros((m, cc), bf16)
    s_ref[m:mp, :] = jnp.zeros((mp - m, (w2 + 2) * cc), bf16)

    xv = x1_ref[...].reshape(B * h1, w1 * cm)
    y = jnp.dot(xv, um_ref[...], preferred_element_type=f32) + ub_ref[...]
    yb = y.astype(bf16)
    half = w2 * cc
    ystk = jnp.concatenate([yb[:, :half], yb[:, half:]], axis=0)

    x2v = x2_ref[...].reshape(B * h2, w2 * cc)
    s_ref[0:m, cc:cc + w2 * cc] = (
        jnp.dot(py_ref[...], ystk, preferred_element_type=f32) +
        jnp.dot(px_ref[...], x2v, preferred_element_type=f32)).astype(bf16)

    accs = []
    for b in range(nb):
        acc = jnp.zeros((m, BLK * cout), f32)
        for ky in range(3):
            acc = acc + jnp.dot(
                s_ref[ky:ky + m, b * BLK * cc:(b * BLK + BLK + 2) * cc],
                w_ref[ky], preferred_element_type=f32)
        accs.append(acc)
    full = jnp.concatenate(accs, axis=1)

    rows = jax.lax.broadcasted_iota(jnp.int32, (m, w2 * cout), 0)
    maskf = (rows % stride < h2).astype(f32)
    fm = full * maskf
    st_ref[0, 0:1, :] = jnp.sum(fm, axis=0, keepdims=True)
    st_ref[0, 1:2, :] = jnp.sum(fm * full, axis=0, keepdims=True)

    ob = full.astype(bf16)
    for i in range(B):
        o_ref[i] = ob[i * stride:i * stride + h2]


def _kernel_b(h2, w2, cin, cout, nb,
              x_ref, sc_ref, sh_ref, px_ref, w_ref, o_ref, st_ref, s_ref):
    f32 = jnp.float32
    bf16 = jnp.bfloat16
    m = B * (h2 + 2)
    mp = m + 8
    stride = h2 + 2

    s_ref[0:m, 0:cin] = jnp.zeros((m, cin), bf16)
    s_ref[0:m, cin + w2 * cin:] = jnp.zeros((m, cin), bf16)
    s_ref[m:mp, :] = jnp.zeros((mp - m, (w2 + 2) * cin), bf16)

    xv = x_ref[...].reshape(B * h2, w2 * cin).astype(f32)
    xv = xv * sc_ref[...] + sh_ref[...]
    xv = jnp.where(xv >= 0.0, xv, NEG * xv).astype(bf16)
    s_ref[0:m, cin:cin + w2 * cin] = jnp.dot(
        px_ref[...], xv, preferred_element_type=f32).astype(bf16)

    accs = []
    for b in range(nb):
        acc = jnp.zeros((m, BLK * cout), f32)
        for ky in range(3):
            acc = acc + jnp.dot(
                s_ref[ky:ky + m, b * BLK * cin:(b * BLK + BLK + 2) * cin],
                w_ref[ky], preferred_element_type=f32)
        accs.append(acc)
    full = jnp.concatenate(accs, axis=1)

    rows = jax.lax.broadcasted_iota(jnp.int32, (m, w2 * cout), 0)
    maskf = (rows % stride < h2).astype(f32)
    fm = full * maskf
    st_ref[0, 0:1, :] = jnp.sum(fm, axis=0, keepdims=True)
    st_ref[0, 1:2, :] = jnp.sum(fm * full, axis=0, keepdims=True)

    ob = full.astype(bf16)
    for i in range(B):
        o_ref[i] = ob[i * stride:i * stride + h2]


def _kernel_c(x_ref, sc_ref, sh_ref, o_ref):
    y = x_ref[...].astype(jnp.float32) * sc_ref[...] + sh_ref[...]
    o_ref[...] = jnp.where(y >= 0.0, y, NEG * y)


def kernel(x1, x2, up_w, up_b, c1_w, c1_b, g1, be1, c2_w, c2_b, g2, be2):
    n, cm, h1, w1 = x1.shape
    h2, w2 = x2.shape[2], x2.shape[3]
    cc = 2 * cm
    cout1 = c1_w.shape[0]
    cout2 = c2_w.shape[0]
    nb = w2 // BLK
    steps = n // B
    m = B * (h2 + 2)
    parallel = pltpu.CompilerParams(dimension_semantics=("parallel",))

    x1s = jnp.transpose(x1, (0, 2, 3, 1)).reshape(n, h1, w1 * cm)
    x1s = x1s.astype(jnp.bfloat16)
    x2p = jnp.pad(jnp.transpose(x2, (0, 2, 3, 1)).astype(jnp.bfloat16),
                  ((0, 0), (0, 0), (0, 0), (0, cc - cm)))
    x2p = x2p.reshape(n, h2, w2 * cc)

    um = _up_mat(up_w, w1, cc)
    ubase = jnp.concatenate([jnp.zeros((cm,), jnp.float32),
                             up_b.astype(jnp.float32)])
    ub = jnp.tile(ubase, 2 * w2).reshape(1, 2 * w2 * cc)
    w1t = _band_tiles(c1_w)
    w2t = _band_tiles(c2_w)

    stride = h2 + 2
    px = _perm([stride * (s // h2) + 1 + (s % h2) for s in range(B * h2)],
               m, B * h2)
    rows_py = []
    for s in range(2 * B * h1):
        di, r = divmod(s, B * h1)
        img, h = divmod(r, h1)
        rows_py.append(stride * img + 1 + 2 * h + di)
    py = _perm(rows_py, m, 2 * B * h1)

    conv1, st1 = pl.pallas_call(
        functools.partial(_kernel_a, h1, w1, cm, h2, w2, cout1, nb),
        grid=(steps,),
        in_specs=[
            pl.BlockSpec((B, h1, w1 * cm), lambda i: (i, 0, 0)),
            pl.BlockSpec((B, h2, w2 * cc), lambda i: (i, 0, 0)),
            pl.BlockSpec((w1 * cm, 2 * w2 * cc), lambda i: (0, 0)),
            pl.BlockSpec((1, 2 * w2 * cc), lambda i: (0, 0)),
            pl.BlockSpec((m, 2 * B * h1), lambda i: (0, 0)),
            pl.BlockSpec((m, B * h2), lambda i: (0, 0)),
            pl.BlockSpec((3, (BLK + 2) * cc, BLK * cout1),
                         lambda i: (0, 0, 0)),
        ],
        out_specs=(pl.BlockSpec((B, h2, w2 * cout1), lambda i: (i, 0, 0)),
                   pl.BlockSpec((1, 2, w2 * cout1), lambda i: (i, 0, 0))),
        out_shape=(jax.ShapeDtypeStruct((n, h2, w2 * cout1), jnp.bfloat16),
                   jax.ShapeDtypeStruct((steps, 2, w2 * cout1), jnp.float32)),
        scratch_shapes=[pltpu.VMEM((m + 8, (w2 + 2) * cc), jnp.bfloat16)],
        compiler_params=parallel,
    )(x1s, x2p, um, ub, py, px, w1t)

    sc1, sh1 = _bn_scale_shift(st1, w2, cout1, n * h2 * w2, g1, be1)

    conv2, st2 = pl.pallas_call(
        functools.partial(_kernel_b, h2, w2, cout1, cout2, nb),
        grid=(steps,),
        in_specs=[
            pl.BlockSpec((B, h2, w2 * cout1), lambda i: (i, 0, 0)),
            pl.BlockSpec((1, w2 * cout1), lambda i: (0, 0)),
            pl.BlockSpec((1, w2 * cout1), lambda i: (0, 0)),
            pl.BlockSpec((m, B * h2), lambda i: (0, 0)),
            pl.BlockSpec((3, (BLK + 2) * cout1, BLK * cout2),
                         lambda i: (0, 0, 0)),
        ],
        out_specs=(pl.BlockSpec((B, h2, w2 * cout2), lambda i: (i, 0, 0)),
                   pl.BlockSpec((1, 2, w2 * cout2), lambda i: (i, 0, 0))),
        out_shape=(jax.ShapeDtypeStruct((n, h2, w2 * cout2), jnp.bfloat16),
                   jax.ShapeDtypeStruct((steps, 2, w2 * cout2), jnp.float32)),
        scratch_shapes=[pltpu.VMEM((m + 8, (w2 + 2) * cout1), jnp.bfloat16)],
        compiler_params=parallel,
    )(conv1, sc1, sh1, px, w2t)

    sc2, sh2 = _bn_scale_shift(st2, w2, cout2, n * h2 * w2, g2, be2)

    out = pl.pallas_call(
        _kernel_c,
        grid=(steps,),
        in_specs=[pl.BlockSpec((B, h2, w2 * cout2), lambda i: (i, 0, 0)),
                  pl.BlockSpec((1, w2 * cout2), lambda i: (0, 0)),
                  pl.BlockSpec((1, w2 * cout2), lambda i: (0, 0))],
        out_specs=pl.BlockSpec((B, h2, w2 * cout2), lambda i: (i, 0, 0)),
        out_shape=jax.ShapeDtypeStruct((n, h2, w2 * cout2), jnp.float32),
        compiler_params=parallel,
    )(conv2, sc2, sh2)

    out = out.reshape(n, h2, w2, cout2)
    return jnp.transpose(out, (0, 3, 1, 2))


# speedup vs baseline: 1.0817x; 1.0817x over previous
import functools

import numpy as np

import jax
import jax.numpy as jnp
from jax.experimental import pallas as pl
from jax.experimental.pallas import tpu as pltpu

NEG = 0.01
EPS = 1e-5
BLK = 8
B = 8


def _band_tiles(w_pt):
    cout, cg = w_pt.shape[0], w_pt.shape[1]
    wt = jnp.transpose(w_pt, (2, 3, 1, 0)).astype(jnp.float32)
    per_ky = []
    for ky in range(3):
        m = jnp.zeros(((BLK + 2) * cg, BLK * cout), jnp.float32)
        for kx in range(3):
            m = m + jnp.kron(jnp.asarray(np.eye(BLK + 2, BLK, k=-kx,
                                                dtype=np.float32)),
                             wt[ky, kx])
        per_ky.append(m)
    return jnp.stack(per_ky).astype(jnp.bfloat16)


def _up_mat(w_pt, w1):
    cm, co = w_pt.shape[0], w_pt.shape[1]
    halves = []
    for di in range(2):
        m = jnp.zeros((w1 * cm, 2 * w1 * co), jnp.float32)
        for dj in range(2):
            sel = np.kron(np.eye(w1, dtype=np.float32),
                          np.eye(1, 2, k=dj, dtype=np.float32))
            m = m + jnp.kron(jnp.asarray(sel),
                             w_pt[:, :, di, dj].astype(jnp.float32))
        halves.append(m)
    return jnp.concatenate(halves, axis=1).astype(jnp.bfloat16)


def _perm(rows, n_rows, n_src):
    p = np.zeros((n_rows, n_src), np.float32)
    p[np.array(rows), np.arange(n_src)] = 1.0
    return jnp.asarray(p, dtype=jnp.bfloat16)


def _bn_scale_shift(st, w, c, count, gamma, beta):
    s = jnp.sum(st, axis=0).reshape(2, w, c).sum(axis=1)
    mean = s[0] / count
    var = s[1] / count - mean * mean
    scale = gamma * jax.lax.rsqrt(var + EPS)
    shift = beta - mean * scale
    return (jnp.tile(scale, w).reshape(1, w * c),
            jnp.tile(shift, w).reshape(1, w * c))


def _kernel_a(h1, w1, cm, h2, w2, co_up, cout, nb,
              x1_ref, x2_ref, um_ref, ub_ref, py_ref, px_ref, wa_ref, wb_ref,
              o_ref, st_ref, sx_ref, sy_ref):
    f32 = jnp.float32
    bf16 = jnp.bfloat16
    m = B * (h2 + 2)
    mp = m + 8
    stride = h2 + 2

    sx_ref[0:m, 0:cm] = jnp.zeros((m, cm), bf16)
    sx_ref[0:m, cm + w2 * cm:] = jnp.zeros((m, cm), bf16)
    sx_ref[m:mp, :] = jnp.zeros((mp - m, (w2 + 2) * cm), bf16)
    sy_ref[0:m, 0:co_up] = jnp.zeros((m, co_up), bf16)
    sy_ref[0:m, co_up + w2 * co_up:] = jnp.zeros((m, co_up), bf16)
    sy_ref[m:mp, :] = jnp.zeros((mp - m, (w2 + 2) * co_up), bf16)

    xv = x1_ref[...].reshape(B * h1, w1 * cm)
    y = jnp.dot(xv, um_ref[...], preferred_element_type=f32) + ub_ref[...]
    yb = y.astype(bf16)
    half = 2 * w1 * co_up
    ystk = jnp.concatenate([yb[:, :half], yb[:, half:]], axis=0)

    sy_ref[0:m, co_up:co_up + w2 * co_up] = jnp.dot(
        py_ref[...], ystk, preferred_element_type=f32).astype(bf16)
    x2v = x2_ref[...].reshape(B * h2, w2 * cm)
    sx_ref[0:m, cm:cm + w2 * cm] = jnp.dot(
        px_ref[...], x2v, preferred_element_type=f32).astype(bf16)

    accs = []
    for b in range(nb):
        acc = jnp.zeros((m, BLK * cout), f32)
        for ky in range(3):
            acc = acc + jnp.dot(
                sx_ref[ky:ky + m, b * BLK * cm:(b * BLK + BLK + 2) * cm],
                wa_ref[ky], preferred_element_type=f32)
            acc = acc + jnp.dot(
                sy_ref[ky:ky + m, b * BLK * co_up:(b * BLK + BLK + 2) * co_up],
                wb_ref[ky], preferred_element_type=f32)
        accs.append(acc)
    full = jnp.concatenate(accs, axis=1)

    rows = jax.lax.broadcasted_iota(jnp.int32, (m, w2 * cout), 0)
    maskf = (rows % stride < h2).astype(f32)
    fm = full * maskf
    st_ref[0, 0:1, :] = jnp.sum(fm, axis=0, keepdims=True)
    st_ref[0, 1:2, :] = jnp.sum(fm * full, axis=0, keepdims=True)

    ob = full.astype(bf16)
    for i in range(B):
        o_ref[i] = ob[i * stride:i * stride + h2]


def _kernel_b(h2, w2, cin, cout, nb,
              x_ref, sc_ref, sh_ref, px_ref, w_ref, o_ref, st_ref, s_ref):
    f32 = jnp.float32
    bf16 = jnp.bfloat16
    m = B * (h2 + 2)
    mp = m + 8
    stride = h2 + 2

    s_ref[0:m, 0:cin] = jnp.zeros((m, cin), bf16)
    s_ref[0:m, cin + w2 * cin:] = jnp.zeros((m, cin), bf16)
    s_ref[m:mp, :] = jnp.zeros((mp - m, (w2 + 2) * cin), bf16)

    xv = x_ref[...].reshape(B * h2, w2 * cin).astype(f32)
    xv = xv * sc_ref[...] + sh_ref[...]
    xv = jnp.where(xv >= 0.0, xv, NEG * xv).astype(bf16)
    s_ref[0:m, cin:cin + w2 * cin] = jnp.dot(
        px_ref[...], xv, preferred_element_type=f32).astype(bf16)

    accs = []
    for b in range(nb):
        acc = jnp.zeros((m, BLK * cout), f32)
        for ky in range(3):
            acc = acc + jnp.dot(
                s_ref[ky:ky + m, b * BLK * cin:(b * BLK + BLK + 2) * cin],
                w_ref[ky], preferred_element_type=f32)
        accs.append(acc)
    full = jnp.concatenate(accs, axis=1)

    rows = jax.lax.broadcasted_iota(jnp.int32, (m, w2 * cout), 0)
    maskf = (rows % stride < h2).astype(f32)
    fm = full * maskf
    st_ref[0, 0:1, :] = jnp.sum(fm, axis=0, keepdims=True)
    st_ref[0, 1:2, :] = jnp.sum(fm * full, axis=0, keepdims=True)

    ob = full.astype(bf16)
    for i in range(B):
        o_ref[i] = ob[i * stride:i * stride + h2]


def _kernel_c(x_ref, sc_ref, sh_ref, o_ref):
    y = x_ref[...].astype(jnp.float32) * sc_ref[...] + sh_ref[...]
    o_ref[...] = jnp.where(y >= 0.0, y, NEG * y)


def kernel(x1, x2, up_w, up_b, c1_w, c1_b, g1, be1, c2_w, c2_b, g2, be2):
    n, cm, h1, w1 = x1.shape
    h2, w2 = x2.shape[2], x2.shape[3]
    co_up = up_w.shape[1]
    cout1 = c1_w.shape[0]
    cout2 = c2_w.shape[0]
    nb = w2 // BLK
    steps = n // B
    m = B * (h2 + 2)
    parallel = pltpu.CompilerParams(dimension_semantics=("parallel",))

    x1s = jnp.transpose(x1, (0, 2, 3, 1)).reshape(n, h1, w1 * cm)
    x1s = x1s.astype(jnp.bfloat16)
    x2s = jnp.transpose(x2, (0, 2, 3, 1)).reshape(n, h2, w2 * cm)
    x2s = x2s.astype(jnp.bfloat16)

    um = _up_mat(up_w, w1)
    ub = jnp.tile(up_b.astype(jnp.float32), 4 * w1).reshape(1, 4 * w1 * co_up)
    wa = _band_tiles(c1_w[:, :cm])
    wb_ = _band_tiles(c1_w[:, cm:])
    w2t = _band_tiles(c2_w)

    stride = h2 + 2
    px = _perm([stride * (s // h2) + 1 + (s % h2) for s in range(B * h2)],
               m, B * h2)
    rows_py = []
    for s in range(2 * B * h1):
        di, r = divmod(s, B * h1)
        img, h = divmod(r, h1)
        rows_py.append(stride * img + 1 + 2 * h + di)
    py = _perm(rows_py, m, 2 * B * h1)

    conv1, st1 = pl.pallas_call(
        functools.partial(_kernel_a, h1, w1, cm, h2, w2, co_up, cout1, nb),
        grid=(steps,),
        in_specs=[
            pl.BlockSpec((B, h1, w1 * cm), lambda i: (i, 0, 0)),
            pl.BlockSpec((B, h2, w2 * cm), lambda i: (i, 0, 0)),
            pl.BlockSpec((w1 * cm, 4 * w1 * co_up), lambda i: (0, 0)),
            pl.BlockSpec((1, 4 * w1 * co_up), lambda i: (0, 0)),
            pl.BlockSpec((m, 2 * B * h1), lambda i: (0, 0)),
            pl.BlockSpec((m, B * h2), lambda i: (0, 0)),
            pl.BlockSpec((3, (BLK + 2) * cm, BLK * cout1),
                         lambda i: (0, 0, 0)),
            pl.BlockSpec((3, (BLK + 2) * co_up, BLK * cout1),
                         lambda i: (0, 0, 0)),
        ],
        out_specs=(pl.BlockSpec((B, h2, w2 * cout1), lambda i: (i, 0, 0)),
                   pl.BlockSpec((1, 2, w2 * cout1), lambda i: (i, 0, 0))),
        out_shape=(jax.ShapeDtypeStruct((n, h2, w2 * cout1), jnp.bfloat16),
                   jax.ShapeDtypeStruct((steps, 2, w2 * cout1), jnp.float32)),
        scratch_shapes=[pltpu.VMEM((m + 8, (w2 + 2) * cm), jnp.bfloat16),
                        pltpu.VMEM((m + 8, (w2 + 2) * co_up), jnp.bfloat16)],
        compiler_params=parallel,
    )(x1s, x2s, um, ub, py, px, wa, wb_)

    sc1, sh1 = _bn_scale_shift(st1, w2, cout1, n * h2 * w2, g1, be1)

    conv2, st2 = pl.pallas_call(
        functools.partial(_kernel_b, h2, w2, cout1, cout2, nb),
        grid=(steps,),
        in_specs=[
            pl.BlockSpec((B, h2, w2 * cout1), lambda i: (i, 0, 0)),
            pl.BlockSpec((1, w2 * cout1), lambda i: (0, 0)),
            pl.BlockSpec((1, w2 * cout1), lambda i: (0, 0)),
            pl.BlockSpec((m, B * h2), lambda i: (0, 0)),
            pl.BlockSpec((3, (BLK + 2) * cout1, BLK * cout2),
                         lambda i: (0, 0, 0)),
        ],
        out_specs=(pl.BlockSpec((B, h2, w2 * cout2), lambda i: (i, 0, 0)),
                   pl.BlockSpec((1, 2, w2 * cout2), lambda i: (i, 0, 0))),
        out_shape=(jax.ShapeDtypeStruct((n, h2, w2 * cout2), jnp.bfloat16),
                   jax.ShapeDtypeStruct((steps, 2, w2 * cout2), jnp.float32)),
        scratch_shapes=[pltpu.VMEM((m + 8, (w2 + 2) * cout1), jnp.bfloat16)],
        compiler_params=parallel,
    )(conv1, sc1, sh1, px, w2t)

    sc2, sh2 = _bn_scale_shift(st2, w2, cout2, n * h2 * w2, g2, be2)

    out = pl.pallas_call(
        _kernel_c,
        grid=(steps,),
        in_specs=[pl.BlockSpec((B, h2, w2 * cout2), lambda i: (i, 0, 0)),
                  pl.BlockSpec((1, w2 * cout2), lambda i: (0, 0)),
                  pl.BlockSpec((1, w2 * cout2), lambda i: (0, 0))],
        out_specs=pl.BlockSpec((B, h2, w2 * cout2), lambda i: (i, 0, 0)),
        out_shape=jax.ShapeDtypeStruct((n, h2, w2 * cout2), jnp.float32),
        compiler_params=parallel,
    )(conv2, sc2, sh2)

    out = out.reshape(n, h2, w2, cout2)
    return jnp.transpose(out, (0, 3, 1, 2))


# speedup vs baseline: 1.1480x; 1.0613x over previous
import functools

import numpy as np

import jax
import jax.numpy as jnp
from jax.experimental import pallas as pl
from jax.experimental.pallas import tpu as pltpu

NEG = 0.01
EPS = 1e-5
BLK = 8
B = 16


def _band_tiles(w_pt):
    cout, cg = w_pt.shape[0], w_pt.shape[1]
    wt = jnp.transpose(w_pt, (2, 3, 1, 0)).astype(jnp.float32)
    per_ky = []
    for ky in range(3):
        m = jnp.zeros(((BLK + 2) * cg, BLK * cout), jnp.float32)
        for kx in range(3):
            m = m + jnp.kron(jnp.asarray(np.eye(BLK + 2, BLK, k=-kx,
                                                dtype=np.float32)),
                             wt[ky, kx])
        per_ky.append(m)
    return jnp.stack(per_ky).astype(jnp.bfloat16)


def _up_mat(w_pt, w1):
    cm, co = w_pt.shape[0], w_pt.shape[1]
    halves = []
    for di in range(2):
        m = jnp.zeros((w1 * cm, 2 * w1 * co), jnp.float32)
        for dj in range(2):
            sel = np.kron(np.eye(w1, dtype=np.float32),
                          np.eye(1, 2, k=dj, dtype=np.float32))
            m = m + jnp.kron(jnp.asarray(sel),
                             w_pt[:, :, di, dj].astype(jnp.float32))
        halves.append(m)
    return jnp.concatenate(halves, axis=1).astype(jnp.bfloat16)


def _perm(rows, n_rows, n_src):
    p = np.zeros((n_rows, n_src), np.float32)
    p[np.array(rows), np.arange(n_src)] = 1.0
    return jnp.asarray(p, dtype=jnp.bfloat16)


def _bn_scale_shift(st, w, c, count, gamma, beta):
    s = jnp.sum(st, axis=0).reshape(2, w, c).sum(axis=1)
    mean = s[0] / count
    var = s[1] / count - mean * mean
    scale = gamma * jax.lax.rsqrt(var + EPS)
    shift = beta - mean * scale
    return (jnp.tile(scale, w).reshape(1, w * c),
            jnp.tile(shift, w).reshape(1, w * c))


def _kernel_a(h1, w1, cm, h2, w2, co_up, cout, nb,
              x1_ref, x2_ref, um_ref, ub_ref, py_ref, px_ref, wa_ref, wb_ref,
              o_ref, st_ref, sx_ref, sy_ref):
    f32 = jnp.float32
    bf16 = jnp.bfloat16
    m = B * (h2 + 2)
    mp = m + 8
    stride = h2 + 2

    sx_ref[0:m, 0:cm] = jnp.zeros((m, cm), bf16)
    sx_ref[0:m, cm + w2 * cm:] = jnp.zeros((m, cm), bf16)
    sx_ref[m:mp, :] = jnp.zeros((mp - m, (w2 + 2) * cm), bf16)
    sy_ref[0:m, 0:co_up] = jnp.zeros((m, co_up), bf16)
    sy_ref[0:m, co_up + w2 * co_up:] = jnp.zeros((m, co_up), bf16)
    sy_ref[m:mp, :] = jnp.zeros((mp - m, (w2 + 2) * co_up), bf16)

    xv = x1_ref[...].reshape(B * h1, w1 * cm)
    y = jnp.dot(xv, um_ref[...], preferred_element_type=f32) + ub_ref[...]
    yb = y.astype(bf16)
    half = 2 * w1 * co_up
    ystk = jnp.concatenate([yb[:, :half], yb[:, half:]], axis=0)

    sy_ref[0:m, co_up:co_up + w2 * co_up] = jnp.dot(
        py_ref[...], ystk, preferred_element_type=f32).astype(bf16)
    x2v = x2_ref[...].reshape(B * h2, w2 * cm)
    sx_ref[0:m, cm:cm + w2 * cm] = jnp.dot(
        px_ref[...], x2v, preferred_element_type=f32).astype(bf16)

    accs = []
    for b in range(nb):
        acc = jnp.zeros((m, BLK * cout), f32)
        for ky in range(3):
            acc = acc + jnp.dot(
                sx_ref[ky:ky + m, b * BLK * cm:(b * BLK + BLK + 2) * cm],
                wa_ref[ky], preferred_element_type=f32)
            acc = acc + jnp.dot(
                sy_ref[ky:ky + m, b * BLK * co_up:(b * BLK + BLK + 2) * co_up],
                wb_ref[ky], preferred_element_type=f32)
        accs.append(acc)
    full = jnp.concatenate(accs, axis=1)

    rows = jax.lax.broadcasted_iota(jnp.int32, (m, w2 * cout), 0)
    maskf = (rows % stride < h2).astype(f32)
    fm = full * maskf
    st_ref[0, 0:1, :] = jnp.sum(fm, axis=0, keepdims=True)
    st_ref[0, 1:2, :] = jnp.sum(fm * full, axis=0, keepdims=True)

    ob = full.astype(bf16)
    for i in range(B):
        o_ref[i] = ob[i * stride:i * stride + h2]


def _kernel_b(h2, w2, cin, cout, nb,
              x_ref, sc_ref, sh_ref, px_ref, w_ref, o_ref, st_ref, s_ref):
    f32 = jnp.float32
    bf16 = jnp.bfloat16
    m = B * (h2 + 2)
    mp = m + 8
    stride = h2 + 2

    s_ref[0:m, 0:cin] = jnp.zeros((m, cin), bf16)
    s_ref[0:m, cin + w2 * cin:] = jnp.zeros((m, cin), bf16)
    s_ref[m:mp, :] = jnp.zeros((mp - m, (w2 + 2) * cin), bf16)

    xv = x_ref[...].reshape(B * h2, w2 * cin).astype(f32)
    xv = xv * sc_ref[...] + sh_ref[...]
    xv = jnp.where(xv >= 0.0, xv, NEG * xv).astype(bf16)
    s_ref[0:m, cin:cin + w2 * cin] = jnp.dot(
        px_ref[...], xv, preferred_element_type=f32).astype(bf16)

    accs = []
    for b in range(nb):
        acc = jnp.zeros((m, BLK * cout), f32)
        for ky in range(3):
            acc = acc + jnp.dot(
                s_ref[ky:ky + m, b * BLK * cin:(b * BLK + BLK + 2) * cin],
                w_ref[ky], preferred_element_type=f32)
        accs.append(acc)
    full = jnp.concatenate(accs, axis=1)

    rows = jax.lax.broadcasted_iota(jnp.int32, (m, w2 * cout), 0)
    maskf = (rows % stride < h2).astype(f32)
    fm = full * maskf
    st_ref[0, 0:1, :] = jnp.sum(fm, axis=0, keepdims=True)
    st_ref[0, 1:2, :] = jnp.sum(fm * full, axis=0, keepdims=True)

    ob = full.astype(bf16)
    for i in range(B):
        o_ref[i] = ob[i * stride:i * stride + h2]


def _kernel_c(x_ref, sc_ref, sh_ref, o_ref):
    y = x_ref[...].astype(jnp.float32) * sc_ref[...] + sh_ref[...]
    o_ref[...] = jnp.where(y >= 0.0, y, NEG * y)


def kernel(x1, x2, up_w, up_b, c1_w, c1_b, g1, be1, c2_w, c2_b, g2, be2):
    n, cm, h1, w1 = x1.shape
    h2, w2 = x2.shape[2], x2.shape[3]
    co_up = up_w.shape[1]
    cout1 = c1_w.shape[0]
    cout2 = c2_w.shape[0]
    nb = w2 // BLK
    steps = n // B
    m = B * (h2 + 2)
    parallel = pltpu.CompilerParams(dimension_semantics=("parallel",))

    x1s = jnp.transpose(x1, (0, 2, 3, 1)).reshape(n, h1, w1 * cm)
    x1s = x1s.astype(jnp.bfloat16)
    x2s = jnp.transpose(x2, (0, 2, 3, 1)).reshape(n, h2, w2 * cm)
    x2s = x2s.astype(jnp.bfloat16)

    um = _up_mat(up_w, w1)
    ub = jnp.tile(up_b.astype(jnp.float32), 4 * w1).reshape(1, 4 * w1 * co_up)
    wa = _band_tiles(c1_w[:, :cm])
    wb_ = _band_tiles(c1_w[:, cm:])
    w2t = _band_tiles(c2_w)

    stride = h2 + 2
    px = _perm([stride * (s // h2) + 1 + (s % h2) for s in range(B * h2)],
               m, B * h2)
    rows_py = []
    for s in range(2 * B * h1):
        di, r = divmod(s, B * h1)
        img, h = divmod(r, h1)
        rows_py.append(stride * img + 1 + 2 * h + di)
    py = _perm(rows_py, m, 2 * B * h1)

    conv1, st1 = pl.pallas_call(
        functools.partial(_kernel_a, h1, w1, cm, h2, w2, co_up, cout1, nb),
        grid=(steps,),
        in_specs=[
            pl.BlockSpec((B, h1, w1 * cm), lambda i: (i, 0, 0)),
            pl.BlockSpec((B, h2, w2 * cm), lambda i: (i, 0, 0)),
            pl.BlockSpec((w1 * cm, 4 * w1 * co_up), lambda i: (0, 0)),
            pl.BlockSpec((1, 4 * w1 * co_up), lambda i: (0, 0)),
            pl.BlockSpec((m, 2 * B * h1), lambda i: (0, 0)),
            pl.BlockSpec((m, B * h2), lambda i: (0, 0)),
            pl.BlockSpec((3, (BLK + 2) * cm, BLK * cout1),
                         lambda i: (0, 0, 0)),
            pl.BlockSpec((3, (BLK + 2) * co_up, BLK * cout1),
                         lambda i: (0, 0, 0)),
        ],
        out_specs=(pl.BlockSpec((B, h2, w2 * cout1), lambda i: (i, 0, 0)),
                   pl.BlockSpec((1, 2, w2 * cout1), lambda i: (i, 0, 0))),
        out_shape=(jax.ShapeDtypeStruct((n, h2, w2 * cout1), jnp.bfloat16),
                   jax.ShapeDtypeStruct((steps, 2, w2 * cout1), jnp.float32)),
        scratch_shapes=[pltpu.VMEM((m + 8, (w2 + 2) * cm), jnp.bfloat16),
                        pltpu.VMEM((m + 8, (w2 + 2) * co_up), jnp.bfloat16)],
        compiler_params=parallel,
    )(x1s, x2s, um, ub, py, px, wa, wb_)

    sc1, sh1 = _bn_scale_shift(st1, w2, cout1, n * h2 * w2, g1, be1)

    conv2, st2 = pl.pallas_call(
        functools.partial(_kernel_b, h2, w2, cout1, cout2, nb),
        grid=(steps,),
        in_specs=[
            pl.BlockSpec((B, h2, w2 * cout1), lambda i: (i, 0, 0)),
            pl.BlockSpec((1, w2 * cout1), lambda i: (0, 0)),
            pl.BlockSpec((1, w2 * cout1), lambda i: (0, 0)),
            pl.BlockSpec((m, B * h2), lambda i: (0, 0)),
            pl.BlockSpec((3, (BLK + 2) * cout1, BLK * cout2),
                         lambda i: (0, 0, 0)),
        ],
        out_specs=(pl.BlockSpec((B, h2, w2 * cout2), lambda i: (i, 0, 0)),
                   pl.BlockSpec((1, 2, w2 * cout2), lambda i: (i, 0, 0))),
        out_shape=(jax.ShapeDtypeStruct((n, h2, w2 * cout2), jnp.bfloat16),
                   jax.ShapeDtypeStruct((steps, 2, w2 * cout2), jnp.float32)),
        scratch_shapes=[pltpu.VMEM((m + 8, (w2 + 2) * cout1), jnp.bfloat16)],
        compiler_params=parallel,
    )(conv1, sc1, sh1, px, w2t)

    sc2, sh2 = _bn_scale_shift(st2, w2, cout2, n * h2 * w2, g2, be2)

    out = pl.pallas_call(
        _kernel_c,
        grid=(steps,),
        in_specs=[pl.BlockSpec((B, h2, w2 * cout2), lambda i: (i, 0, 0)),
                  pl.BlockSpec((1, w2 * cout2), lambda i: (0, 0)),
                  pl.BlockSpec((1, w2 * cout2), lambda i: (0, 0))],
        out_specs=pl.BlockSpec((B, h2, w2 * cout2), lambda i: (i, 0, 0)),
        out_shape=jax.ShapeDtypeStruct((n, h2, w2 * cout2), jnp.float32),
        compiler_params=parallel,
    )(conv2, sc2, sh2)

    out = out.reshape(n, h2, w2, cout2)
    return jnp.transpose(out, (0, 3, 1, 2))
